# baseline (device time: 234751 ns/iter reference)
import jax
import jax.numpy as jnp
from jax import lax
from jax.experimental import pallas as pl
from jax.experimental.pallas import tpu as pltpu

M_GLOBAL = 8192
D = 4096
M_SHARD = 4096
M_QUARTER = 2048
EPS = 1e-6

SIZES = [256] * 7 + [192, 64]
OFFS = [sum(SIZES[:i]) for i in range(len(SIZES))]
NCH = len(SIZES)
SLOT = 256

_CompilerParams = getattr(pltpu, "CompilerParams", None) or getattr(
    pltpu, "TPUCompilerParams"
)


def kernel(partial, gamma):
    gamma2d = gamma.reshape(1, D)

    def body(
        partial_ref, gamma_ref, out_ref,
        xload, local, send_x, recv_x, send_y,
        load_peer_sems, load_local_sems, store_mine_sems,
        send_x_sems, recv_x_sems, send_y_sems, recv_y_sems,
    ):
        my_x = lax.axis_index("x")
        my_y = lax.axis_index("y")
        peer_x = (1 - my_x, my_y)
        peer_y = (my_x, 1 - my_y)

        barrier = pltpu.get_barrier_semaphore()
        for nbr in (peer_x, peer_y):
            pl.semaphore_signal(
                barrier, inc=1, device_id=nbr,
                device_id_type=pl.DeviceIdType.MESH,
            )
        pl.semaphore_wait(barrier, 2)

        qstart = my_x * M_SHARD + my_y * M_QUARTER
        pstart = (1 - my_x) * M_SHARD + my_y * M_QUARTER

        def load_peer(c):
            off, sz = OFFS[c], SIZES[c]
            return pltpu.make_async_copy(
                partial_ref.at[0, pl.ds(pstart + off, sz), :],
                xload.at[pl.ds((c % 2) * SLOT, sz), :],
                load_peer_sems.at[c % 2],
            )

        def load_local(c):
            off, sz = OFFS[c], SIZES[c]
            return pltpu.make_async_copy(
                partial_ref.at[0, pl.ds(qstart + off, sz), :],
                local.at[pl.ds((c % 2) * SLOT, sz), :],
                load_local_sems.at[c % 2],
            )

        def xdesc(c):
            off, sz = OFFS[c], SIZES[c]
            return pltpu.make_async_remote_copy(
                src_ref=send_x.at[pl.ds(off, sz), :],
                dst_ref=recv_x.at[pl.ds(off, sz), :],
                send_sem=send_x_sems.at[c], recv_sem=recv_x_sems.at[c],
                device_id=peer_x, device_id_type=pl.DeviceIdType.MESH,
            )

        def ydesc(c):
            off, sz = OFFS[c], SIZES[c]
            return pltpu.make_async_remote_copy(
                src_ref=send_y.at[pl.ds((c % 4) * SLOT, sz), :],
                dst_ref=out_ref.at[pl.ds(my_y * M_QUARTER + off, sz), :],
                send_sem=send_y_sems.at[c % 4], recv_sem=recv_y_sems.at[c % 4],
                device_id=peer_y, device_id_type=pl.DeviceIdType.MESH,
            )

        def store_mine(c):
            off, sz = OFFS[c], SIZES[c]
            return pltpu.make_async_copy(
                send_y.at[pl.ds((c % 4) * SLOT, sz), :],
                out_ref.at[pl.ds(my_y * M_QUARTER + off, sz), :],
                store_mine_sems.at[c % 4],
            )

        load_peer(0).start()
        load_peer(1).start()
        for c in range(NCH):
            off, sz, s = OFFS[c], SIZES[c], (c % 2) * SLOT
            load_peer(c).wait()
            send_x[off:off + sz] = xload[s:s + sz].astype(jnp.bfloat16)
            xdesc(c).start()
            if c + 2 < NCH:
                load_peer(c + 2).start()

        load_local(0).start()
        load_local(1).start()

        for c in range(NCH):
            off, sz, s = OFFS[c], SIZES[c], (c % 2) * SLOT
            sy = (c % 4) * SLOT
            load_local(c).wait()
            xdesc(c).wait_recv()
            if c >= 4:
                ydesc(c - 4).wait_send()
                store_mine(c - 4).wait()
            ssum = local[s:s + sz] + recv_x[off:off + sz].astype(jnp.float32)
            ms = jnp.mean(ssum * ssum, axis=-1, keepdims=True)
            o = ssum * lax.rsqrt(ms + EPS) * gamma_ref[...]
            send_y[sy:sy + sz] = o.astype(jnp.bfloat16)

            ydesc(c).start()
            store_mine(c).start()
            if c + 2 < NCH:
                load_local(c + 2).start()

            if c >= 1:
                ydesc(c - 1).wait_recv()

        last = NCH - 1
        ydesc(last).wait_recv()

        for k in range(NCH):
            xdesc(k).wait_send()
        for k in range(last - 3, last + 1):
            ydesc(k).wait_send()
            store_mine(k).wait()

    return pl.pallas_call(
        body,
        out_shape=jax.ShapeDtypeStruct((M_SHARD, D), jnp.bfloat16),
        in_specs=[
            pl.BlockSpec(memory_space=pl.ANY),
            pl.BlockSpec(memory_space=pltpu.VMEM),
        ],
        out_specs=pl.BlockSpec(memory_space=pltpu.MemorySpace.HBM),
        scratch_shapes=[
            pltpu.VMEM((2 * SLOT, D), jnp.float32),
            pltpu.VMEM((2 * SLOT, D), jnp.float32),
            pltpu.VMEM((M_QUARTER, D), jnp.bfloat16),
            pltpu.VMEM((M_QUARTER, D), jnp.bfloat16),
            pltpu.VMEM((4 * SLOT, D), jnp.bfloat16),
            pltpu.SemaphoreType.DMA((2,)),
            pltpu.SemaphoreType.DMA((2,)),
            pltpu.SemaphoreType.DMA((4,)),
            pltpu.SemaphoreType.DMA((NCH,)),
            pltpu.SemaphoreType.DMA((NCH,)),
            pltpu.SemaphoreType.DMA((4,)),
            pltpu.SemaphoreType.DMA((4,)),
        ],
        compiler_params=_CompilerParams(
            collective_id=0,
            vmem_limit_bytes=63 * 1024 * 1024,
        ),
    )(partial, gamma2d)
